# baseline (speedup 1.0000x reference)
import functools

import jax
import jax.numpy as jnp
from jax import lax
from jax.experimental import pallas as pl
from jax.experimental.pallas import tpu as pltpu
from jax.experimental.pallas import tpu_sc as plsc

NUM_USERS = 100000
NUM_MOVIES = 100000
EMBED_DIM = 128
INPUT_DIM = 64
BATCH = 16384
MEAN_RATING = 3.5

_NC = 2
_NS = 16
_NW = _NC * _NS
_CHUNK = 128
_DEPTH = 3
_PHASES = 2
_PB = BATCH // _PHASES
_BLK = 1024


def _gather_call(user_idx, movie_idx, user_embed, movie_embed,
                 ubias1d, mbias1d):
    n = _PB
    b_per_w = n // _NW
    nchunk = b_per_w // _CHUNK
    mesh = plsc.VectorSubcoreMesh(core_axis_name="c", subcore_axis_name="s")

    @functools.partial(
        pl.kernel,
        out_type=(
            jax.ShapeDtypeStruct((n, EMBED_DIM), jnp.float32),
            jax.ShapeDtypeStruct((n, EMBED_DIM), jnp.float32),
            jax.ShapeDtypeStruct((n,), jnp.float32),
            jax.ShapeDtypeStruct((n,), jnp.float32),
        ),
        mesh=mesh,
        compiler_params=pltpu.CompilerParams(use_tc_tiling_on_sc=False),
        scratch_types=[
            pltpu.VMEM((nchunk, _CHUNK), jnp.int32),
            pltpu.VMEM((nchunk, _CHUNK), jnp.int32),
        ] + [
            pltpu.VMEM((_CHUNK, EMBED_DIM), jnp.float32)
            for _ in range(_DEPTH)
        ] + [
            pltpu.VMEM((b_per_w,), jnp.float32),
            pltpu.VMEM((b_per_w,), jnp.float32),
        ] + [pltpu.SemaphoreType.DMA for _ in range(2 * _DEPTH)]
        + [pltpu.SemaphoreType.DMA],
    )
    def gather_k(uidx_hbm, midx_hbm, utab_hbm, mtab_hbm, ubias_hbm, mbias_hbm,
                 uout, mout, ubout, mbout,
                 idx_u, idx_m, *rest):
        bufs = rest[:_DEPTH]
        ub, mb = rest[_DEPTH], rest[_DEPTH + 1]
        sems = rest[_DEPTH + 2:_DEPTH + 2 + _DEPTH]
        ssems = rest[_DEPTH + 2 + _DEPTH:_DEPTH + 2 + 2 * _DEPTH]
        bsem = rest[_DEPTH + 2 + 2 * _DEPTH]

        wid = lax.axis_index("s") * _NC + lax.axis_index("c")
        base = wid * b_per_w

        pltpu.sync_copy(uidx_hbm.at[wid], idx_u)
        pltpu.sync_copy(midx_hbm.at[wid], idx_m)

        bias_copies = []
        for c in range(nchunk):
            bias_copies.append(pltpu.async_copy(
                ubias_hbm.at[idx_u.at[c]],
                ub.at[pl.ds(c * _CHUNK, _CHUNK)], bsem))
            bias_copies.append(pltpu.async_copy(
                mbias_hbm.at[idx_m.at[c]],
                mb.at[pl.ds(c * _CHUNK, _CHUNK)], bsem))

        steps = []
        for c in range(nchunk):
            steps.append((utab_hbm, idx_u.at[c], uout, c))
        for c in range(nchunk):
            steps.append((mtab_hbm, idx_m.at[c], mout, c))

        nsteps = len(steps)
        copies = [None] * nsteps
        stores = [None] * nsteps

        def _start_store(s):
            _, _, s_out, s_c = steps[s]
            copies[s].wait()
            stores[s] = pltpu.async_copy(
                bufs[s % _DEPTH],
                s_out.at[pl.ds(base + s_c * _CHUNK, _CHUNK)],
                ssems[s % _DEPTH])

        for t, (tab, idx, _, _c) in enumerate(steps):
            if t >= _DEPTH:
                stores[t - _DEPTH].wait()
            copies[t] = pltpu.async_copy(tab.at[idx], bufs[t % _DEPTH],
                                         sems[t % _DEPTH])
            if t >= _DEPTH - 1:
                _start_store(t - (_DEPTH - 1))
        for s in range(max(nsteps - (_DEPTH - 1), 0), nsteps):
            _start_store(s)
        for s in range(max(nsteps - _DEPTH, 0), nsteps):
            stores[s].wait()

        for cp in bias_copies:
            cp.wait()
        pltpu.sync_copy(ub, ubout.at[pl.ds(base, b_per_w)])
        pltpu.sync_copy(mb, mbout.at[pl.ds(base, b_per_w)])

    uidx = user_idx.astype(jnp.int32).reshape(_NW, nchunk, _CHUNK)
    midx = movie_idx.astype(jnp.int32).reshape(_NW, nchunk, _CHUNK)
    return gather_k(uidx, midx, user_embed, movie_embed, ubias1d, mbias1d)


_SCALE = 0.08838834764831845


def _mlp_body(nxt, u, m, ub, mb, w1, b1, w2, b2, w3r, b3, o):
    uu = u[...]
    mm = m[...]
    h = jax.lax.dot_general(nxt[...], w1[0:INPUT_DIM, :],
                            (((0,), (0,)), ((), ())),
                            preferred_element_type=jnp.float32)
    hu = jnp.dot(uu, w1[INPUT_DIM:INPUT_DIM + EMBED_DIM, :],
                 preferred_element_type=jnp.float32)
    hm = jnp.dot(mm, w1[INPUT_DIM + EMBED_DIM:INPUT_DIM + 2 * EMBED_DIM, :],
                 preferred_element_type=jnp.float32)
    hi = jnp.dot(uu * mm, w1[INPUT_DIM + 2 * EMBED_DIM:, :],
                 preferred_element_type=jnp.float32)
    h = h + _SCALE * (hu + hm) + (_SCALE * _SCALE) * hi
    h = jnp.maximum(h + b1[...], 0.0)
    h = jnp.maximum(jnp.dot(h, w2[...], preferred_element_type=jnp.float32)
                    + b2[...], 0.0)
    ht = jax.lax.transpose(h, (1, 0))
    s = jnp.dot(w3r[...], ht, preferred_element_type=jnp.float32)
    o[0] = s + b3[0, 0] + ub[0] + mb[0] + MEAN_RATING


def _mlp_call(phase, numeric_x_t, u_rows, m_rows, ub, mb, W1, b1, W2, b2,
              W3, b3, interpret=False):
    blk = _BLK
    nblk = _PB // blk
    grid = (nblk,)
    row = lambda i: (i, 0)
    fixed = lambda i: (0, 0)
    return pl.pallas_call(
        _mlp_body,
        grid=grid,
        in_specs=[
            pl.BlockSpec((INPUT_DIM, blk), lambda i: (0, phase * nblk + i)),
            pl.BlockSpec((blk, EMBED_DIM), row),
            pl.BlockSpec((blk, EMBED_DIM), row),
            pl.BlockSpec((1, 1, blk), lambda i: (i, 0, 0)),
            pl.BlockSpec((1, 1, blk), lambda i: (i, 0, 0)),
            pl.BlockSpec((INPUT_DIM + 3 * EMBED_DIM, 64), fixed),
            pl.BlockSpec((1, 64), fixed),
            pl.BlockSpec((64, 32), fixed),
            pl.BlockSpec((1, 32), fixed),
            pl.BlockSpec((1, 32), fixed),
            pl.BlockSpec((1, 1), fixed),
        ],
        out_specs=pl.BlockSpec((1, 1, blk), lambda i: (i, 0, 0)),
        out_shape=jax.ShapeDtypeStruct((nblk, 1, blk), jnp.float32),
        interpret=interpret,
    )(numeric_x_t, u_rows, m_rows, ub, mb, W1, b1, W2, b2, W3, b3)


def kernel(numeric_x, user_x, movie_x, user_embed, movie_embed,
           user_bias, movie_bias, W1, b1, W2, b2, W3, b3):
    nxt = numeric_x.T
    ub1d = user_bias.reshape(-1)
    mb1d = movie_bias.reshape(-1)
    b1r = b1.reshape(1, 64)
    b2r = b2.reshape(1, 32)
    w3r = W3.reshape(1, 32)
    b3r = b3.reshape(1, 1)

    nblk = _PB // _BLK
    outs = []
    for p in range(_PHASES):
        sl = slice(p * _PB, (p + 1) * _PB)
        u_rows, m_rows, ub, mb = _gather_call(
            user_x[sl], movie_x[sl], user_embed, movie_embed, ub1d, mb1d)
        outs.append(_mlp_call(
            p, nxt, u_rows, m_rows,
            ub.reshape(nblk, 1, _BLK), mb.reshape(nblk, 1, _BLK),
            W1, b1r, W2, b2r, w3r, b3r))
    return jnp.concatenate(outs, axis=0).reshape(BATCH, 1)


# speedup vs baseline: 1.0487x; 1.0487x over previous
import functools

import jax
import jax.numpy as jnp
from jax import lax
from jax.experimental import pallas as pl
from jax.experimental.pallas import tpu as pltpu
from jax.experimental.pallas import tpu_sc as plsc

NUM_USERS = 100000
NUM_MOVIES = 100000
EMBED_DIM = 128
INPUT_DIM = 64
BATCH = 16384
MEAN_RATING = 3.5

_NC = 2
_NS = 16
_NW = _NC * _NS
_B_PER_W = BATCH // _NW
_CHUNK = 128
_NCHUNK = _B_PER_W // _CHUNK
_DEPTH = 3


def _gather_call(user_idx, movie_idx, user_embed, movie_embed,
                 user_bias, movie_bias):
    mesh = plsc.VectorSubcoreMesh(core_axis_name="c", subcore_axis_name="s")

    @functools.partial(
        pl.kernel,
        out_type=(
            jax.ShapeDtypeStruct((BATCH, EMBED_DIM), jnp.float32),
            jax.ShapeDtypeStruct((BATCH, EMBED_DIM), jnp.float32),
            jax.ShapeDtypeStruct((BATCH,), jnp.float32),
            jax.ShapeDtypeStruct((BATCH,), jnp.float32),
        ),
        mesh=mesh,
        compiler_params=pltpu.CompilerParams(use_tc_tiling_on_sc=False),
        scratch_types=[
            pltpu.VMEM((_NCHUNK, _CHUNK), jnp.int32),
            pltpu.VMEM((_NCHUNK, _CHUNK), jnp.int32),
        ] + [
            pltpu.VMEM((_CHUNK, EMBED_DIM), jnp.float32)
            for _ in range(_DEPTH)
        ] + [
            pltpu.VMEM((_B_PER_W,), jnp.float32),
            pltpu.VMEM((_B_PER_W,), jnp.float32),
        ] + [pltpu.SemaphoreType.DMA for _ in range(2 * _DEPTH)]
        + [pltpu.SemaphoreType.DMA],
    )
    def gather_k(uidx_hbm, midx_hbm, utab_hbm, mtab_hbm, ubias_hbm, mbias_hbm,
                 uout, mout, ubout, mbout,
                 idx_u, idx_m, *rest):
        bufs = rest[:_DEPTH]
        ub, mb = rest[_DEPTH], rest[_DEPTH + 1]
        sems = rest[_DEPTH + 2:_DEPTH + 2 + _DEPTH]
        ssems = rest[_DEPTH + 2 + _DEPTH:_DEPTH + 2 + 2 * _DEPTH]
        bsem = rest[_DEPTH + 2 + 2 * _DEPTH]

        wid = lax.axis_index("s") * _NC + lax.axis_index("c")
        base = wid * _B_PER_W

        pltpu.sync_copy(uidx_hbm.at[wid], idx_u)
        pltpu.sync_copy(midx_hbm.at[wid], idx_m)

        bias_copies = []
        for c in range(_NCHUNK):
            bias_copies.append(pltpu.async_copy(
                ubias_hbm.at[idx_u.at[c]],
                ub.at[pl.ds(c * _CHUNK, _CHUNK)], bsem))
            bias_copies.append(pltpu.async_copy(
                mbias_hbm.at[idx_m.at[c]],
                mb.at[pl.ds(c * _CHUNK, _CHUNK)], bsem))

        steps = []
        for c in range(_NCHUNK):
            steps.append((utab_hbm, idx_u.at[c], uout, c))
        for c in range(_NCHUNK):
            steps.append((mtab_hbm, idx_m.at[c], mout, c))

        n = len(steps)
        copies = [None] * n
        stores = [None] * n

        def _start_store(s):
            _, _, s_out, s_c = steps[s]
            copies[s].wait()
            stores[s] = pltpu.async_copy(
                bufs[s % _DEPTH],
                s_out.at[pl.ds(base + s_c * _CHUNK, _CHUNK)],
                ssems[s % _DEPTH])

        for t, (tab, idx, _, _c) in enumerate(steps):
            if t >= _DEPTH:
                stores[t - _DEPTH].wait()
            copies[t] = pltpu.async_copy(tab.at[idx], bufs[t % _DEPTH],
                                         sems[t % _DEPTH])
            if t >= _DEPTH - 1:
                _start_store(t - (_DEPTH - 1))
        for s in range(n - (_DEPTH - 1), n):
            _start_store(s)
        for s in range(n - _DEPTH, n):
            stores[s].wait()

        for cp in bias_copies:
            cp.wait()
        pltpu.sync_copy(ub, ubout.at[pl.ds(base, _B_PER_W)])
        pltpu.sync_copy(mb, mbout.at[pl.ds(base, _B_PER_W)])

    uidx = user_idx.astype(jnp.int32).reshape(_NW, _NCHUNK, _CHUNK)
    midx = movie_idx.astype(jnp.int32).reshape(_NW, _NCHUNK, _CHUNK)
    pad = (-NUM_USERS) % 128
    ub1d = jnp.pad(user_bias, ((0, pad), (0, 0))).reshape(-1)
    mb1d = jnp.pad(movie_bias, ((0, pad), (0, 0))).reshape(-1)
    return gather_k(uidx, midx, user_embed, movie_embed, ub1d, mb1d)


_SCALE = 0.08838834764831845


def _mlp_body(nxt, u, m, ub, mb, w1, b1, w2, b2, w3r, b3, o):
    blk = u.shape[0]
    i = pl.program_id(0)
    uu = u[...]
    mm = m[...]
    h = jax.lax.dot_general(nxt[:, pl.ds(i * blk, blk)], w1[0:INPUT_DIM, :],
                            (((0,), (0,)), ((), ())),
                            preferred_element_type=jnp.float32)
    hu = jnp.dot(uu, w1[INPUT_DIM:INPUT_DIM + EMBED_DIM, :],
                 preferred_element_type=jnp.float32)
    hm = jnp.dot(mm, w1[INPUT_DIM + EMBED_DIM:INPUT_DIM + 2 * EMBED_DIM, :],
                 preferred_element_type=jnp.float32)
    hi = jnp.dot(uu * mm, w1[INPUT_DIM + 2 * EMBED_DIM:, :],
                 preferred_element_type=jnp.float32)
    h = h + _SCALE * (hu + hm) + (_SCALE * _SCALE) * hi
    h = jnp.maximum(h + b1[...], 0.0)
    h = jnp.maximum(jnp.dot(h, w2[...], preferred_element_type=jnp.float32)
                    + b2[...], 0.0)
    ht = jax.lax.transpose(h, (1, 0))
    s = jnp.dot(w3r[...], ht, preferred_element_type=jnp.float32)
    o[0] = s + b3[0, 0] + ub[0] + mb[0] + MEAN_RATING


def _mlp_call(numeric_x_t, u_rows, m_rows, ub, mb, W1, b1, W2, b2, W3, b3,
              interpret=False):
    blk = 1024
    grid = (BATCH // blk,)
    row = lambda i: (i, 0)
    fixed = lambda i: (0, 0)
    return pl.pallas_call(
        _mlp_body,
        grid=grid,
        in_specs=[
            pl.BlockSpec((INPUT_DIM, BATCH), lambda i: (0, 0)),
            pl.BlockSpec((blk, EMBED_DIM), row),
            pl.BlockSpec((blk, EMBED_DIM), row),
            pl.BlockSpec((1, 1, blk), lambda i: (i, 0, 0)),
            pl.BlockSpec((1, 1, blk), lambda i: (i, 0, 0)),
            pl.BlockSpec((INPUT_DIM + 3 * EMBED_DIM, 64), fixed),
            pl.BlockSpec((1, 64), fixed),
            pl.BlockSpec((64, 32), fixed),
            pl.BlockSpec((1, 32), fixed),
            pl.BlockSpec((1, 32), fixed),
            pl.BlockSpec((1, 1), fixed),
        ],
        out_specs=pl.BlockSpec((1, 1, blk), lambda i: (i, 0, 0)),
        out_shape=jax.ShapeDtypeStruct((BATCH // blk, 1, blk), jnp.float32),
        interpret=interpret,
    )(numeric_x_t, u_rows, m_rows, ub, mb, W1, b1, W2, b2, W3, b3)


def kernel(numeric_x, user_x, movie_x, user_embed, movie_embed,
           user_bias, movie_bias, W1, b1, W2, b2, W3, b3):
    u_rows, m_rows, ub, mb = _gather_call(
        user_x, movie_x, user_embed, movie_embed, user_bias, movie_bias)

    blk = 1024
    out = _mlp_call(
        numeric_x.T, u_rows, m_rows,
        ub.reshape(BATCH // blk, 1, blk), mb.reshape(BATCH // blk, 1, blk),
        W1, b1.reshape(1, 64), W2, b2.reshape(1, 32), W3.reshape(1, 32),
        b3.reshape(1, 1))
    return out.reshape(BATCH, 1)


# speedup vs baseline: 1.1367x; 1.0838x over previous
import functools

import jax
import jax.numpy as jnp
from jax import lax
from jax.experimental import pallas as pl
from jax.experimental.pallas import tpu as pltpu
from jax.experimental.pallas import tpu_sc as plsc

NUM_USERS = 100000
NUM_MOVIES = 100000
EMBED_DIM = 128
INPUT_DIM = 64
BATCH = 16384
MEAN_RATING = 3.5

_NC = 2
_NS = 16
_NW = _NC * _NS
_B_PER_W = BATCH // _NW
_CHUNK = 128
_NCHUNK = _B_PER_W // _CHUNK
_DEPTH = 3


def _gather_call(user_idx, movie_idx, user_embed, movie_embed,
                 user_bias, movie_bias):
    mesh = plsc.VectorSubcoreMesh(core_axis_name="c", subcore_axis_name="s")

    @functools.partial(
        pl.kernel,
        out_type=(
            jax.ShapeDtypeStruct((BATCH, EMBED_DIM), jnp.float32),
            jax.ShapeDtypeStruct((BATCH, EMBED_DIM), jnp.float32),
            jax.ShapeDtypeStruct((BATCH,), jnp.float32),
            jax.ShapeDtypeStruct((BATCH,), jnp.float32),
        ),
        mesh=mesh,
        compiler_params=pltpu.CompilerParams(use_tc_tiling_on_sc=False),
        scratch_types=[
            pltpu.VMEM((_NCHUNK, _CHUNK), jnp.int32),
            pltpu.VMEM((_NCHUNK, _CHUNK), jnp.int32),
        ] + [
            pltpu.VMEM((_CHUNK, EMBED_DIM), jnp.float32)
            for _ in range(_DEPTH)
        ] + [
            pltpu.VMEM((_B_PER_W,), jnp.float32),
            pltpu.VMEM((_B_PER_W,), jnp.float32),
        ] + [pltpu.SemaphoreType.DMA for _ in range(2 * _DEPTH)]
        + [pltpu.SemaphoreType.DMA],
    )
    def gather_k(uidx_hbm, midx_hbm, utab_hbm, mtab_hbm, ubias_hbm, mbias_hbm,
                 uout, mout, ubout, mbout,
                 idx_u, idx_m, *rest):
        bufs = rest[:_DEPTH]
        ub, mb = rest[_DEPTH], rest[_DEPTH + 1]
        sems = rest[_DEPTH + 2:_DEPTH + 2 + _DEPTH]
        ssems = rest[_DEPTH + 2 + _DEPTH:_DEPTH + 2 + 2 * _DEPTH]
        bsem = rest[_DEPTH + 2 + 2 * _DEPTH]

        wid = lax.axis_index("s") * _NC + lax.axis_index("c")
        base = wid * _B_PER_W

        pltpu.sync_copy(uidx_hbm.at[wid], idx_u)
        pltpu.sync_copy(midx_hbm.at[wid], idx_m)

        bias_copies = []
        for c in range(_NCHUNK):
            bias_copies.append(pltpu.async_copy(
                ubias_hbm.at[idx_u.at[c]],
                ub.at[pl.ds(c * _CHUNK, _CHUNK)], bsem))
            bias_copies.append(pltpu.async_copy(
                mbias_hbm.at[idx_m.at[c]],
                mb.at[pl.ds(c * _CHUNK, _CHUNK)], bsem))

        steps = []
        for c in range(_NCHUNK):
            steps.append((utab_hbm, idx_u.at[c], uout, c))
        for c in range(_NCHUNK):
            steps.append((mtab_hbm, idx_m.at[c], mout, c))

        n = len(steps)
        copies = [None] * n
        stores = [None] * n

        def _start_store(s):
            _, _, s_out, s_c = steps[s]
            copies[s].wait()
            stores[s] = pltpu.async_copy(
                bufs[s % _DEPTH],
                s_out.at[pl.ds(base + s_c * _CHUNK, _CHUNK)],
                ssems[s % _DEPTH])

        for t, (tab, idx, _, _c) in enumerate(steps):
            if t >= _DEPTH:
                stores[t - _DEPTH].wait()
            copies[t] = pltpu.async_copy(tab.at[idx], bufs[t % _DEPTH],
                                         sems[t % _DEPTH])
            if t >= _DEPTH - 1:
                _start_store(t - (_DEPTH - 1))
        for s in range(n - (_DEPTH - 1), n):
            _start_store(s)
        for s in range(n - _DEPTH, n):
            stores[s].wait()

        for cp in bias_copies:
            cp.wait()
        pltpu.sync_copy(ub, ubout.at[pl.ds(base, _B_PER_W)])
        pltpu.sync_copy(mb, mbout.at[pl.ds(base, _B_PER_W)])

    uidx = user_idx.astype(jnp.int32).reshape(_NW, _NCHUNK, _CHUNK)
    midx = movie_idx.astype(jnp.int32).reshape(_NW, _NCHUNK, _CHUNK)
    pad = (-NUM_USERS) % 128
    ub1d = jnp.pad(user_bias, ((0, pad), (0, 0))).reshape(-1)
    mb1d = jnp.pad(movie_bias, ((0, pad), (0, 0))).reshape(-1)
    return gather_k(uidx, midx, user_embed, movie_embed, ub1d, mb1d)


_SCALE = 0.08838834764831845


def _mlp_body(nxt, u, m, ub, mb, w1, b1, w2, b2, w3r, b3, o):
    blk = u.shape[0]
    i = pl.program_id(0)
    uu = u[...]
    mm = m[...]
    h = jax.lax.dot_general(nxt[:, pl.ds(i * blk, blk)], w1[0:INPUT_DIM, :],
                            (((0,), (0,)), ((), ())),
                            preferred_element_type=jnp.float32)
    hu = jnp.dot(uu, w1[INPUT_DIM:INPUT_DIM + EMBED_DIM, :],
                 preferred_element_type=jnp.float32)
    hm = jnp.dot(mm, w1[INPUT_DIM + EMBED_DIM:INPUT_DIM + 2 * EMBED_DIM, :],
                 preferred_element_type=jnp.float32)
    hi = jnp.dot(uu * mm, w1[INPUT_DIM + 2 * EMBED_DIM:, :],
                 preferred_element_type=jnp.float32)
    h = h + _SCALE * (hu + hm) + (_SCALE * _SCALE) * hi
    h = jnp.maximum(h + b1[...], 0.0)
    h = jnp.maximum(jnp.dot(h, w2[...], preferred_element_type=jnp.float32)
                    + b2[...], 0.0)
    ht = jax.lax.transpose(h, (1, 0))
    s = jnp.dot(w3r[...], ht, preferred_element_type=jnp.float32)
    o[0] = s + b3[0, 0] + ub[0] + mb[0] + MEAN_RATING


def _mlp_call(numeric_x_t, u_rows, m_rows, ub, mb, W1, b1, W2, b2, W3, b3,
              interpret=False):
    blk = 2048
    grid = (BATCH // blk,)
    row = lambda i: (i, 0)
    fixed = lambda i: (0, 0)
    return pl.pallas_call(
        _mlp_body,
        grid=grid,
        in_specs=[
            pl.BlockSpec((INPUT_DIM, BATCH), lambda i: (0, 0)),
            pl.BlockSpec((blk, EMBED_DIM), row),
            pl.BlockSpec((blk, EMBED_DIM), row),
            pl.BlockSpec((1, 1, blk), lambda i: (i, 0, 0)),
            pl.BlockSpec((1, 1, blk), lambda i: (i, 0, 0)),
            pl.BlockSpec((INPUT_DIM + 3 * EMBED_DIM, 64), fixed),
            pl.BlockSpec((1, 64), fixed),
            pl.BlockSpec((64, 32), fixed),
            pl.BlockSpec((1, 32), fixed),
            pl.BlockSpec((1, 32), fixed),
            pl.BlockSpec((1, 1), fixed),
        ],
        out_specs=pl.BlockSpec((1, 1, blk), lambda i: (i, 0, 0)),
        out_shape=jax.ShapeDtypeStruct((BATCH // blk, 1, blk), jnp.float32),
        interpret=interpret,
    )(numeric_x_t, u_rows, m_rows, ub, mb, W1, b1, W2, b2, W3, b3)


def kernel(numeric_x, user_x, movie_x, user_embed, movie_embed,
           user_bias, movie_bias, W1, b1, W2, b2, W3, b3):
    u_rows, m_rows, ub, mb = _gather_call(
        user_x, movie_x, user_embed, movie_embed, user_bias, movie_bias)

    blk = 2048
    out = _mlp_call(
        numeric_x.T, u_rows, m_rows,
        ub.reshape(BATCH // blk, 1, blk), mb.reshape(BATCH // blk, 1, blk),
        W1, b1.reshape(1, 64), W2, b2.reshape(1, 32), W3.reshape(1, 32),
        b3.reshape(1, 1))
    return out.reshape(BATCH, 1)


# speedup vs baseline: 1.1836x; 1.0413x over previous
import functools

import jax
import jax.numpy as jnp
from jax import lax
from jax.experimental import pallas as pl
from jax.experimental.pallas import tpu as pltpu
from jax.experimental.pallas import tpu_sc as plsc

NUM_USERS = 100000
NUM_MOVIES = 100000
EMBED_DIM = 128
INPUT_DIM = 64
BATCH = 16384
MEAN_RATING = 3.5

_NC = 2
_NS = 16
_NW = _NC * _NS
_B_PER_W = BATCH // _NW
_CHUNK = 128
_NCHUNK = _B_PER_W // _CHUNK
_DEPTH = 3


def _gather_call(user_idx, movie_idx, user_embed, movie_embed,
                 user_bias, movie_bias):
    mesh = plsc.VectorSubcoreMesh(core_axis_name="c", subcore_axis_name="s")

    @functools.partial(
        pl.kernel,
        out_type=(
            jax.ShapeDtypeStruct((BATCH, EMBED_DIM), jnp.float32),
            jax.ShapeDtypeStruct((BATCH, EMBED_DIM), jnp.float32),
            jax.ShapeDtypeStruct((BATCH,), jnp.float32),
            jax.ShapeDtypeStruct((BATCH,), jnp.float32),
        ),
        mesh=mesh,
        compiler_params=pltpu.CompilerParams(use_tc_tiling_on_sc=False),
        scratch_types=[
            pltpu.VMEM((_NCHUNK, _CHUNK), jnp.int32),
            pltpu.VMEM((_NCHUNK, _CHUNK), jnp.int32),
        ] + [
            pltpu.VMEM((_CHUNK, EMBED_DIM), jnp.float32)
            for _ in range(_DEPTH)
        ] + [
            pltpu.VMEM((_B_PER_W,), jnp.float32),
            pltpu.VMEM((_B_PER_W,), jnp.float32),
        ] + [pltpu.SemaphoreType.DMA for _ in range(2 * _DEPTH)]
        + [pltpu.SemaphoreType.DMA],
    )
    def gather_k(uidx_hbm, midx_hbm, utab_hbm, mtab_hbm, ubias_hbm, mbias_hbm,
                 uout, mout, ubout, mbout,
                 idx_u, idx_m, *rest):
        bufs = rest[:_DEPTH]
        ub, mb = rest[_DEPTH], rest[_DEPTH + 1]
        sems = rest[_DEPTH + 2:_DEPTH + 2 + _DEPTH]
        ssems = rest[_DEPTH + 2 + _DEPTH:_DEPTH + 2 + 2 * _DEPTH]
        bsem = rest[_DEPTH + 2 + 2 * _DEPTH]

        wid = lax.axis_index("s") * _NC + lax.axis_index("c")
        base = wid * _B_PER_W

        pltpu.sync_copy(uidx_hbm.at[wid], idx_u)
        pltpu.sync_copy(midx_hbm.at[wid], idx_m)

        bias_copies = []
        for c in range(_NCHUNK):
            bias_copies.append(pltpu.async_copy(
                ubias_hbm.at[idx_u.at[c]],
                ub.at[pl.ds(c * _CHUNK, _CHUNK)], bsem))
            bias_copies.append(pltpu.async_copy(
                mbias_hbm.at[idx_m.at[c]],
                mb.at[pl.ds(c * _CHUNK, _CHUNK)], bsem))

        steps = []
        for c in range(_NCHUNK):
            steps.append((utab_hbm, idx_u.at[c], uout, c))
        for c in range(_NCHUNK):
            steps.append((mtab_hbm, idx_m.at[c], mout, c))

        n = len(steps)
        copies = [None] * n
        stores = [None] * n

        def _start_store(s):
            _, _, s_out, s_c = steps[s]
            copies[s].wait()
            stores[s] = pltpu.async_copy(
                bufs[s % _DEPTH],
                s_out.at[pl.ds(base + s_c * _CHUNK, _CHUNK)],
                ssems[s % _DEPTH])

        for t, (tab, idx, _, _c) in enumerate(steps):
            if t >= _DEPTH:
                stores[t - _DEPTH].wait()
            copies[t] = pltpu.async_copy(tab.at[idx], bufs[t % _DEPTH],
                                         sems[t % _DEPTH])
            if t >= _DEPTH - 1:
                _start_store(t - (_DEPTH - 1))
        for s in range(n - (_DEPTH - 1), n):
            _start_store(s)
        for s in range(n - _DEPTH, n):
            stores[s].wait()

        for cp in bias_copies:
            cp.wait()
        pltpu.sync_copy(ub, ubout.at[pl.ds(base, _B_PER_W)])
        pltpu.sync_copy(mb, mbout.at[pl.ds(base, _B_PER_W)])

    uidx = user_idx.astype(jnp.int32).reshape(_NW, _NCHUNK, _CHUNK)
    midx = movie_idx.astype(jnp.int32).reshape(_NW, _NCHUNK, _CHUNK)
    pad = (-NUM_USERS) % 128
    ub1d = jnp.pad(user_bias, ((0, pad), (0, 0))).reshape(-1)
    mb1d = jnp.pad(movie_bias, ((0, pad), (0, 0))).reshape(-1)
    return gather_k(uidx, midx, user_embed, movie_embed, ub1d, mb1d)


_SCALE = 0.08838834764831845


def _mlp_body(nxt, u, m, ub, mb, w1, b1, w2, b2, w3r, b3, o):
    blk = u.shape[0]
    i = pl.program_id(0)
    uu = u[...]
    mm = m[...]
    h = jax.lax.dot_general(nxt[:, pl.ds(i * blk, blk)], w1[0:INPUT_DIM, :],
                            (((0,), (0,)), ((), ())),
                            preferred_element_type=jnp.float32)
    hu = jnp.dot(uu, w1[INPUT_DIM:INPUT_DIM + EMBED_DIM, :],
                 preferred_element_type=jnp.float32)
    hm = jnp.dot(mm, w1[INPUT_DIM + EMBED_DIM:INPUT_DIM + 2 * EMBED_DIM, :],
                 preferred_element_type=jnp.float32)
    hi = jnp.dot(uu * mm, w1[INPUT_DIM + 2 * EMBED_DIM:, :],
                 preferred_element_type=jnp.float32)
    h = h + _SCALE * (hu + hm) + (_SCALE * _SCALE) * hi
    h = jnp.maximum(h + b1[...], 0.0)
    h = jnp.maximum(jnp.dot(h, w2[...], preferred_element_type=jnp.float32)
                    + b2[...], 0.0)
    ht = jax.lax.transpose(h, (1, 0))
    s = jnp.dot(w3r[...], ht, preferred_element_type=jnp.float32)
    o[0] = s + b3[0, 0] + ub[0] + mb[0] + MEAN_RATING


def _mlp_call(numeric_x_t, u_rows, m_rows, ub, mb, W1, b1, W2, b2, W3, b3,
              interpret=False):
    blk = 4096
    grid = (BATCH // blk,)
    row = lambda i: (i, 0)
    fixed = lambda i: (0, 0)
    return pl.pallas_call(
        _mlp_body,
        grid=grid,
        in_specs=[
            pl.BlockSpec((INPUT_DIM, BATCH), lambda i: (0, 0)),
            pl.BlockSpec((blk, EMBED_DIM), row),
            pl.BlockSpec((blk, EMBED_DIM), row),
            pl.BlockSpec((1, 1, blk), lambda i: (i, 0, 0)),
            pl.BlockSpec((1, 1, blk), lambda i: (i, 0, 0)),
            pl.BlockSpec((INPUT_DIM + 3 * EMBED_DIM, 64), fixed),
            pl.BlockSpec((1, 64), fixed),
            pl.BlockSpec((64, 32), fixed),
            pl.BlockSpec((1, 32), fixed),
            pl.BlockSpec((1, 32), fixed),
            pl.BlockSpec((1, 1), fixed),
        ],
        out_specs=pl.BlockSpec((1, 1, blk), lambda i: (i, 0, 0)),
        out_shape=jax.ShapeDtypeStruct((BATCH // blk, 1, blk), jnp.float32),
        interpret=interpret,
    )(numeric_x_t, u_rows, m_rows, ub, mb, W1, b1, W2, b2, W3, b3)


def kernel(numeric_x, user_x, movie_x, user_embed, movie_embed,
           user_bias, movie_bias, W1, b1, W2, b2, W3, b3):
    u_rows, m_rows, ub, mb = _gather_call(
        user_x, movie_x, user_embed, movie_embed, user_bias, movie_bias)

    blk = 4096
    out = _mlp_call(
        numeric_x.T, u_rows, m_rows,
        ub.reshape(BATCH // blk, 1, blk), mb.reshape(BATCH // blk, 1, blk),
        W1, b1.reshape(1, 64), W2, b2.reshape(1, 32), W3.reshape(1, 32),
        b3.reshape(1, 1))
    return out.reshape(BATCH, 1)
